# baseline (speedup 1.0000x reference)
import functools

import jax
import jax.numpy as jnp
from jax import lax
from jax.experimental import pallas as pl
from jax.experimental.pallas import tpu as pltpu
from jax.experimental.pallas import tpu_sc as plsc

NC = 2
NS = 16
L = 16


def _tile_body(bpw, d_latent, uid_hbm, iid_hbm, user_hbm, item_hbm, out_hbm,
               idx_u, idx_i, rows_u, rows_i, out_v, sem_u, sem_i):
    wid = lax.axis_index("s") * NC + lax.axis_index("c")
    base = wid * bpw

    pltpu.sync_copy(uid_hbm.at[pl.ds(base, bpw)], idx_u)
    pltpu.sync_copy(iid_hbm.at[pl.ds(base, bpw)], idx_i)
    cu = pltpu.async_copy(user_hbm.at[idx_u], rows_u, sem_u)
    ci = pltpu.async_copy(item_hbm.at[idx_i], rows_i, sem_i)
    cu.wait()
    ci.wait()

    lane = jnp.arange(L, dtype=jnp.int32)

    def blk(r, carry):
        row_idx = r * L + lane
        acc = jnp.zeros((L,), jnp.float32)
        for d in range(d_latent):
            col = jnp.full((L,), d, jnp.int32)
            gu = plsc.load_gather(rows_u, [row_idx, col])
            gi = plsc.load_gather(rows_i, [row_idx, col])
            acc = acc + gu * gi
        out_v[pl.ds(r * L, L)] = acc
        return carry

    lax.fori_loop(0, bpw // L, blk, 0, unroll=2)

    pltpu.sync_copy(out_v, out_hbm.at[pl.ds(base, bpw)])


@jax.jit
def _run(uid, iid, user_table, item_table):
    b = uid.shape[0]
    d_latent = user_table.shape[1]
    nw = NC * NS
    bpw = b // nw
    mesh = plsc.VectorSubcoreMesh(
        core_axis_name="c", subcore_axis_name="s",
        num_cores=NC, num_subcores=NS)
    body = functools.partial(_tile_body, bpw, d_latent)
    return pl.kernel(
        body,
        out_type=jax.ShapeDtypeStruct((b,), jnp.float32),
        mesh=mesh,
        compiler_params=pltpu.CompilerParams(needs_layout_passes=False,
                                             use_tc_tiling_on_sc=False),
        scratch_types=[
            pltpu.VMEM((bpw,), jnp.int32),
            pltpu.VMEM((bpw,), jnp.int32),
            pltpu.VMEM((bpw, d_latent), jnp.float32),
            pltpu.VMEM((bpw, d_latent), jnp.float32),
            pltpu.VMEM((bpw,), jnp.float32),
            pltpu.SemaphoreType.DMA,
            pltpu.SemaphoreType.DMA,
        ],
    )(uid, iid, user_table, item_table)


def kernel(x, user_table, item_table):
    uid = x[:, 0]
    iid = x[:, 1]
    return _run(uid, iid, user_table, item_table)
